# baseline (speedup 1.0000x reference)
import functools

import jax
import jax.numpy as jnp
from jax import lax
from jax.experimental import pallas as pl
from jax.experimental.pallas import tpu as pltpu
from jax.experimental.pallas import tpu_sc as plsc

_NC = 2
_NS = 16
_NW = _NC * _NS
_CH = 128
_NG = 128
_BN = 1000


def _ceil_to(a, m):
    return (a + m - 1) // m * m


@functools.lru_cache(None)
def _make_deg(nck, npad):
    rpt = npad // _NS
    mesh = plsc.VectorSubcoreMesh(core_axis_name="c", subcore_axis_name="s")

    def body(eidx_hbm, zo_hbm, out_hbm, dstv, onesv, acc, sem):
        c = lax.axis_index("c")
        s = lax.axis_index("s")
        wid = c * _NS + s
        pltpu.sync_copy(zo_hbm.at[pl.ds(s * rpt, rpt), :],
                        acc.at[pl.ds(s * rpt, rpt), :])
        pltpu.sync_copy(zo_hbm.at[pl.ds(npad, _CH), :], onesv)
        pltpu.sync_copy(eidx_hbm.at[1, pl.ds(wid * nck, nck), :], dstv)
        plsc.subcore_barrier()

        def fire(j, carry):
            pltpu.async_copy(onesv, acc.at[dstv.at[j]], sem, add=True)
            return carry

        lax.fori_loop(0, nck, fire, 0)

        def drain(j, carry):
            pltpu.make_async_copy(onesv, acc.at[dstv.at[j]], sem).wait()
            return carry

        lax.fori_loop(0, nck, drain, 0)
        plsc.subcore_barrier()
        pltpu.sync_copy(acc.at[pl.ds(s * rpt, rpt), :],
                        out_hbm.at[c, pl.ds(s * rpt, rpt), :])

    return pl.kernel(
        body,
        out_type=jax.ShapeDtypeStruct((_NC, npad, 16), jnp.float32),
        mesh=mesh,
        scratch_types=[
            pltpu.VMEM((nck, _CH), jnp.int32),
            pltpu.VMEM((_CH, 16), jnp.float32),
            pltpu.VMEM_SHARED((npad, 16), jnp.float32),
            pltpu.SemaphoreType.DMA,
        ],
        compiler_params=pltpu.CompilerParams(use_tc_tiling_on_sc=False),
    )


@functools.lru_cache(None)
def _make_agg(n, d, nck, npad):
    rpt = npad // _NS
    mesh = plsc.VectorSubcoreMesh(core_axis_name="c", subcore_axis_name="s")
    assert nck % 4 == 0 and nck >= 8

    def body(h_hbm, eidx_hbm, out_hbm, srcv, dstv,
             r0, r1, r2, acc, hstage, g0, g1, g2, s0, s1, s2):
        c = lax.axis_index("c")
        s = lax.axis_index("s")
        wid = c * _NS + s
        rows = (r0, r1, r2)
        gsem = (g0, g1, g2)
        ssem = (s0, s1, s2)
        start = jnp.minimum(s * rpt, n - rpt)
        pltpu.sync_copy(h_hbm.at[pl.ds(start, rpt), :],
                        acc.at[pl.ds(start, rpt), :])
        pltpu.sync_copy(h_hbm.at[pl.ds(start, rpt), :],
                        hstage.at[pl.ds(start, rpt), :])
        pltpu.sync_copy(eidx_hbm.at[0, pl.ds(wid * nck, nck), :], srcv)
        pltpu.sync_copy(eidx_hbm.at[1, pl.ds(wid * nck, nck), :], dstv)
        plsc.subcore_barrier()

        def g_start(j, p):
            pltpu.async_copy(hstage.at[srcv.at[j]], rows[p], gsem[p])

        def g_wait(j, p):
            pltpu.make_async_copy(hstage.at[srcv.at[j]], rows[p],
                                  gsem[p]).wait()

        def s_start(j, p):
            pltpu.async_copy(rows[p], acc.at[dstv.at[j]], ssem[p], add=True)

        def s_wait(j, p):
            pltpu.make_async_copy(rows[p], acc.at[dstv.at[j]],
                                  ssem[p]).wait()

        g_start(0, 0)
        g_start(1, 1)
        g_wait(0, 0)
        s_start(0, 0)
        g_start(2, 2)
        m3 = (nck - 3) // 3

        def core(jj, carry):
            for p in range(3):
                j = 3 * jj + 1 + p
                bb = (1 + p) % 3
                g_wait(j, bb)
                s_start(j, bb)
                s_wait(j - 1, p % 3)
                g_start(j + 2, p % 3)
            return carry

        lax.fori_loop(0, m3, core, 0)
        for j in range(3 * m3 + 1, nck):
            g_wait(j, j % 3)
            s_start(j, j % 3)
            s_wait(j - 1, (j - 1) % 3)
            nj = j + 2
            if nj < nck and nj > 3 * m3 + 2:
                g_start(nj, nj % 3)
        s_wait(nck - 1, (nck - 1) % 3)
        plsc.subcore_barrier()
        pltpu.sync_copy(acc.at[pl.ds(s * rpt, rpt), :],
                        out_hbm.at[c, pl.ds(s * rpt, rpt), :])

    return pl.kernel(
        body,
        out_type=jax.ShapeDtypeStruct((_NC, npad, d), jnp.float32),
        mesh=mesh,
        scratch_types=[
            pltpu.VMEM((nck, _CH), jnp.int32),
            pltpu.VMEM((nck, _CH), jnp.int32),
            pltpu.VMEM((_CH, d), jnp.float32),
            pltpu.VMEM((_CH, d), jnp.float32),
            pltpu.VMEM((_CH, d), jnp.float32),
            pltpu.VMEM_SHARED((npad, d), jnp.float32),
            pltpu.VMEM_SHARED((npad, d), jnp.float32),
            pltpu.SemaphoreType.DMA,
            pltpu.SemaphoreType.DMA,
            pltpu.SemaphoreType.DMA,
            pltpu.SemaphoreType.DMA,
            pltpu.SemaphoreType.DMA,
            pltpu.SemaphoreType.DMA,
        ],
        compiler_params=pltpu.CompilerParams(use_tc_tiling_on_sc=False),
    )


def _matmul_tc(x, w):
    n, din = x.shape
    hid = w.shape[1]
    nblk = n // _BN

    def body(x_ref, w_ref, o_ref):
        o_ref[...] = jnp.dot(x_ref[...], w_ref[...],
                             preferred_element_type=jnp.float32)

    return pl.pallas_call(
        body,
        grid=(nblk,),
        in_specs=[
            pl.BlockSpec((_BN, din), lambda i: (i, 0)),
            pl.BlockSpec((din, hid), lambda i: (0, 0)),
        ],
        out_specs=pl.BlockSpec((_BN, hid), lambda i: (i, 0)),
        out_shape=jax.ShapeDtypeStruct((n, hid), jnp.float32),
    )(x, w)


def _scale_tc(h, dg):
    n, hid = h.shape
    nblk = n // _BN

    def body(h_ref, dg_ref, o_ref):
        dinv = lax.rsqrt(dg_ref[...] + 1.0)
        o_ref[...] = h_ref[...] * dinv

    return pl.pallas_call(
        body,
        grid=(nblk,),
        in_specs=[
            pl.BlockSpec((_BN, hid), lambda i: (i, 0)),
            pl.BlockSpec((_BN, 1), lambda i: (i, 0)),
        ],
        out_specs=pl.BlockSpec((_BN, hid), lambda i: (i, 0)),
        out_shape=jax.ShapeDtypeStruct((n, hid), jnp.float32),
    )(h, dg)


def _layer2_tc(ag, h, dg, b, w):
    n, hid = h.shape
    hid2 = w.shape[1]
    nblk = n // _BN

    def body(ag_ref, h_ref, dg_ref, b_ref, w_ref, o_ref):
        dinv = lax.rsqrt(dg_ref[...] + 1.0)
        agv = ag_ref[...]
        pre = (agv[0] + agv[1] - h_ref[...]) * dinv + b_ref[...]
        pre = jnp.maximum(pre, 0.0)
        o_ref[...] = jnp.dot(pre, w_ref[...],
                             preferred_element_type=jnp.float32) * dinv

    return pl.pallas_call(
        body,
        grid=(nblk,),
        in_specs=[
            pl.BlockSpec((_NC, _BN, hid), lambda i: (0, i, 0)),
            pl.BlockSpec((_BN, hid), lambda i: (i, 0)),
            pl.BlockSpec((_BN, 1), lambda i: (i, 0)),
            pl.BlockSpec((1, hid), lambda i: (0, 0)),
            pl.BlockSpec((hid, hid2), lambda i: (0, 0)),
        ],
        out_specs=pl.BlockSpec((_BN, hid2), lambda i: (i, 0)),
        out_shape=jax.ShapeDtypeStruct((n, hid2), jnp.float32),
    )(ag, h, dg, b, w)


def _final_tc(ag, h, dg, b, wfct, bfc, batch):
    n, hid = h.shape
    nblk = n // _BN

    def body(ag_ref, h_ref, dg_ref, b_ref, wfct_ref, bfc_ref, batch_ref,
             o_ref, sacc, cacc):
        i = pl.program_id(0)

        @pl.when(i == 0)
        def _():
            sacc[...] = jnp.zeros_like(sacc)
            cacc[...] = jnp.zeros_like(cacc)

        dinv = lax.rsqrt(dg_ref[...] + 1.0)
        agv = ag_ref[...]
        pre = (agv[0] + agv[1] - h_ref[...]) * dinv + b_ref[...]
        pre = jnp.maximum(pre, 0.0)
        y = jnp.sum(pre * wfct_ref[...], axis=1, keepdims=True) + bfc_ref[...]
        gids = lax.broadcasted_iota(jnp.int32, (_BN, _NG), 1)
        m = (batch_ref[...] == gids).astype(jnp.float32)
        sacc[...] += jnp.sum(m * y, axis=0)[None, :]
        cacc[...] += jnp.sum(m, axis=0)[None, :]

        @pl.when(i == nblk - 1)
        def _():
            o_ref[...] = sacc[...] / jnp.maximum(cacc[...], 1.0)

    return pl.pallas_call(
        body,
        grid=(nblk,),
        in_specs=[
            pl.BlockSpec((_NC, _BN, hid), lambda i: (0, i, 0)),
            pl.BlockSpec((_BN, hid), lambda i: (i, 0)),
            pl.BlockSpec((_BN, 1), lambda i: (i, 0)),
            pl.BlockSpec((1, hid), lambda i: (0, 0)),
            pl.BlockSpec((1, hid), lambda i: (0, 0)),
            pl.BlockSpec((1, 1), lambda i: (0, 0)),
            pl.BlockSpec((_BN, 1), lambda i: (i, 0)),
        ],
        out_specs=pl.BlockSpec((1, _NG), lambda i: (0, 0)),
        out_shape=jax.ShapeDtypeStruct((1, _NG), jnp.float32),
        scratch_shapes=[
            pltpu.VMEM((1, _NG), jnp.float32),
            pltpu.VMEM((1, _NG), jnp.float32),
        ],
    )(ag, h, dg, b, wfct, bfc, batch)


def kernel(x, edge_index, batch, W1, b1, W2, b2, Wfc, bfc):
    n, _ = x.shape
    hid = W1.shape[1]
    e = edge_index.shape[1]
    npad = _ceil_to((n + _NS - 1) // _NS, 8) * _NS
    if npad == n:
        npad += 8 * _NS
    ec = _ceil_to(e, _CH)
    nrows = _ceil_to(ec // _CH, _NW * 4)
    nck = nrows // _NW

    pad_e = nrows * _CH - e
    if pad_e:
        ar = jnp.arange(pad_e, dtype=jnp.int32)
        pad = jnp.stack([ar % n, n + ar % (npad - n)])
        eidx = jnp.concatenate([edge_index, pad.reshape(2, pad_e)], axis=1)
    else:
        eidx = edge_index
    eidx = eidx.reshape(2, nrows, _CH)

    zo = jnp.concatenate([jnp.zeros((npad, 16), jnp.float32),
                          jnp.ones((_CH, 16), jnp.float32)])
    degp = _make_deg(nck, npad)(eidx, zo)
    dg = (degp[0, :, 0] + degp[1, :, 0]).reshape(npad, 1)[:n]

    h1u = _matmul_tc(x, W1)
    h1 = _scale_tc(h1u, dg)
    ag1 = _make_agg(n, hid, nck, npad)(h1, eidx)
    h2 = _layer2_tc(ag1, h1, dg, b1.reshape(1, hid), W2)
    ag2 = _make_agg(n, hid, nck, npad)(h2, eidx)
    pooled = _final_tc(ag2, h2, dg, b2.reshape(1, hid),
                       Wfc.reshape(1, hid), bfc.reshape(1, 1),
                       batch.reshape(n, 1))
    return pooled.reshape(_NG, 1)


# speedup vs baseline: 1.0612x; 1.0612x over previous
import functools

import jax
import jax.numpy as jnp
from jax import lax
from jax.experimental import pallas as pl
from jax.experimental.pallas import tpu as pltpu
from jax.experimental.pallas import tpu_sc as plsc

_NC = 2
_NS = 16
_NW = _NC * _NS
_CH = 128
_NG = 128
_BN = 1000


def _ceil_to(a, m):
    return (a + m - 1) // m * m


@functools.lru_cache(None)
def _make_deg(nck, npad):
    rpt = npad // _NS
    mesh = plsc.VectorSubcoreMesh(core_axis_name="c", subcore_axis_name="s")

    def body(eidx_hbm, zo_hbm, out_hbm, dstv, onesv, acc, sem):
        c = lax.axis_index("c")
        s = lax.axis_index("s")
        wid = c * _NS + s
        pltpu.sync_copy(zo_hbm.at[pl.ds(s * rpt, rpt), :],
                        acc.at[pl.ds(s * rpt, rpt), :])
        pltpu.sync_copy(zo_hbm.at[pl.ds(npad, _CH), :], onesv)
        pltpu.sync_copy(eidx_hbm.at[1, pl.ds(wid * nck, nck), :], dstv)
        plsc.subcore_barrier()

        def fire(j, carry):
            pltpu.async_copy(onesv, acc.at[dstv.at[j]], sem, add=True)
            return carry

        lax.fori_loop(0, nck, fire, 0)

        def drain(j, carry):
            pltpu.make_async_copy(onesv, acc.at[dstv.at[j]], sem).wait()
            return carry

        lax.fori_loop(0, nck, drain, 0)
        plsc.subcore_barrier()
        pltpu.sync_copy(acc.at[pl.ds(s * rpt, rpt), :],
                        out_hbm.at[c, pl.ds(s * rpt, rpt), :])

    return pl.kernel(
        body,
        out_type=jax.ShapeDtypeStruct((_NC, npad, 16), jnp.float32),
        mesh=mesh,
        scratch_types=[
            pltpu.VMEM((nck, _CH), jnp.int32),
            pltpu.VMEM((_CH, 16), jnp.float32),
            pltpu.VMEM_SHARED((npad, 16), jnp.float32),
            pltpu.SemaphoreType.DMA,
        ],
        compiler_params=pltpu.CompilerParams(use_tc_tiling_on_sc=False),
    )


@functools.lru_cache(None)
def _make_agg(n, d, nck, npad):
    rpt = npad // _NS
    mesh = plsc.VectorSubcoreMesh(core_axis_name="c", subcore_axis_name="s")
    assert nck % 4 == 0 and nck >= 8

    def body(h_hbm, eidx_hbm, out_hbm, srcv, dstv,
             r0, r1, r2, r3, acc, g0, g1, g2, g3, s0, s1, s2, s3):
        c = lax.axis_index("c")
        s = lax.axis_index("s")
        wid = c * _NS + s
        rows = (r0, r1, r2, r3)
        gsem = (g0, g1, g2, g3)
        ssem = (s0, s1, s2, s3)
        start = jnp.minimum(s * rpt, n - rpt)
        pltpu.sync_copy(h_hbm.at[pl.ds(start, rpt), :],
                        acc.at[pl.ds(start, rpt), :])
        pltpu.sync_copy(eidx_hbm.at[0, pl.ds(wid * nck, nck), :], srcv)
        pltpu.sync_copy(eidx_hbm.at[1, pl.ds(wid * nck, nck), :], dstv)
        plsc.subcore_barrier()

        def g_start(j, p):
            pltpu.async_copy(h_hbm.at[srcv.at[j]], rows[p], gsem[p])

        def g_wait(j, p):
            pltpu.make_async_copy(h_hbm.at[srcv.at[j]], rows[p],
                                  gsem[p]).wait()

        def s_start(j, p):
            pltpu.async_copy(rows[p], acc.at[dstv.at[j]], ssem[p], add=True)

        def s_wait(j, p):
            pltpu.make_async_copy(rows[p], acc.at[dstv.at[j]],
                                  ssem[p]).wait()

        for p in range(4):
            g_start(p, p)
        g_wait(0, 0)
        s_start(0, 0)
        g_wait(1, 1)
        s_start(1, 1)

        def core(jj, carry):
            for p in range(4):
                j = 4 * jj + p
                s_wait(j - 4, p)
                g_start(j, p)
                g_wait(j - 2, (p + 2) % 4)
                s_start(j - 2, (p + 2) % 4)
            return carry

        lax.fori_loop(1, nck // 4, core, 0)
        g_wait(nck - 2, 2)
        s_start(nck - 2, 2)
        g_wait(nck - 1, 3)
        s_start(nck - 1, 3)
        s_wait(nck - 4, 0)
        s_wait(nck - 3, 1)
        s_wait(nck - 2, 2)
        s_wait(nck - 1, 3)
        plsc.subcore_barrier()
        pltpu.sync_copy(acc.at[pl.ds(s * rpt, rpt), :],
                        out_hbm.at[c, pl.ds(s * rpt, rpt), :])

    return pl.kernel(
        body,
        out_type=jax.ShapeDtypeStruct((_NC, npad, d), jnp.float32),
        mesh=mesh,
        scratch_types=[
            pltpu.VMEM((nck, _CH), jnp.int32),
            pltpu.VMEM((nck, _CH), jnp.int32),
            pltpu.VMEM((_CH, d), jnp.float32),
            pltpu.VMEM((_CH, d), jnp.float32),
            pltpu.VMEM((_CH, d), jnp.float32),
            pltpu.VMEM((_CH, d), jnp.float32),
            pltpu.VMEM_SHARED((npad, d), jnp.float32),
            pltpu.SemaphoreType.DMA,
            pltpu.SemaphoreType.DMA,
            pltpu.SemaphoreType.DMA,
            pltpu.SemaphoreType.DMA,
            pltpu.SemaphoreType.DMA,
            pltpu.SemaphoreType.DMA,
            pltpu.SemaphoreType.DMA,
            pltpu.SemaphoreType.DMA,
        ],
        compiler_params=pltpu.CompilerParams(use_tc_tiling_on_sc=False),
    )


def _matmul_tc(x, w):
    n, din = x.shape
    hid = w.shape[1]
    nblk = n // _BN

    def body(x_ref, w_ref, o_ref):
        o_ref[...] = jnp.dot(x_ref[...], w_ref[...],
                             preferred_element_type=jnp.float32)

    return pl.pallas_call(
        body,
        grid=(nblk,),
        in_specs=[
            pl.BlockSpec((_BN, din), lambda i: (i, 0)),
            pl.BlockSpec((din, hid), lambda i: (0, 0)),
        ],
        out_specs=pl.BlockSpec((_BN, hid), lambda i: (i, 0)),
        out_shape=jax.ShapeDtypeStruct((n, hid), jnp.float32),
    )(x, w)


def _scale_tc(h, dg):
    n, hid = h.shape
    nblk = n // _BN

    def body(h_ref, dg_ref, o_ref):
        dinv = lax.rsqrt(dg_ref[...] + 1.0)
        o_ref[...] = h_ref[...] * dinv

    return pl.pallas_call(
        body,
        grid=(nblk,),
        in_specs=[
            pl.BlockSpec((_BN, hid), lambda i: (i, 0)),
            pl.BlockSpec((_BN, 1), lambda i: (i, 0)),
        ],
        out_specs=pl.BlockSpec((_BN, hid), lambda i: (i, 0)),
        out_shape=jax.ShapeDtypeStruct((n, hid), jnp.float32),
    )(h, dg)


def _layer2_tc(ag, h, dg, b, w):
    n, hid = h.shape
    hid2 = w.shape[1]
    nblk = n // _BN

    def body(ag_ref, h_ref, dg_ref, b_ref, w_ref, o_ref):
        dinv = lax.rsqrt(dg_ref[...] + 1.0)
        agv = ag_ref[...]
        pre = (agv[0] + agv[1] - h_ref[...]) * dinv + b_ref[...]
        pre = jnp.maximum(pre, 0.0)
        o_ref[...] = jnp.dot(pre, w_ref[...],
                             preferred_element_type=jnp.float32) * dinv

    return pl.pallas_call(
        body,
        grid=(nblk,),
        in_specs=[
            pl.BlockSpec((_NC, _BN, hid), lambda i: (0, i, 0)),
            pl.BlockSpec((_BN, hid), lambda i: (i, 0)),
            pl.BlockSpec((_BN, 1), lambda i: (i, 0)),
            pl.BlockSpec((1, hid), lambda i: (0, 0)),
            pl.BlockSpec((hid, hid2), lambda i: (0, 0)),
        ],
        out_specs=pl.BlockSpec((_BN, hid2), lambda i: (i, 0)),
        out_shape=jax.ShapeDtypeStruct((n, hid2), jnp.float32),
    )(ag, h, dg, b, w)


def _final_tc(ag, h, dg, b, wfct, bfc, batch):
    n, hid = h.shape
    nblk = n // _BN

    def body(ag_ref, h_ref, dg_ref, b_ref, wfct_ref, bfc_ref, batch_ref,
             o_ref, sacc, cacc):
        i = pl.program_id(0)

        @pl.when(i == 0)
        def _():
            sacc[...] = jnp.zeros_like(sacc)
            cacc[...] = jnp.zeros_like(cacc)

        dinv = lax.rsqrt(dg_ref[...] + 1.0)
        agv = ag_ref[...]
        pre = (agv[0] + agv[1] - h_ref[...]) * dinv + b_ref[...]
        pre = jnp.maximum(pre, 0.0)
        y = jnp.sum(pre * wfct_ref[...], axis=1, keepdims=True) + bfc_ref[...]
        gids = lax.broadcasted_iota(jnp.int32, (_BN, _NG), 1)
        m = (batch_ref[...] == gids).astype(jnp.float32)
        sacc[...] += jnp.sum(m * y, axis=0)[None, :]
        cacc[...] += jnp.sum(m, axis=0)[None, :]

        @pl.when(i == nblk - 1)
        def _():
            o_ref[...] = sacc[...] / jnp.maximum(cacc[...], 1.0)

    return pl.pallas_call(
        body,
        grid=(nblk,),
        in_specs=[
            pl.BlockSpec((_NC, _BN, hid), lambda i: (0, i, 0)),
            pl.BlockSpec((_BN, hid), lambda i: (i, 0)),
            pl.BlockSpec((_BN, 1), lambda i: (i, 0)),
            pl.BlockSpec((1, hid), lambda i: (0, 0)),
            pl.BlockSpec((1, hid), lambda i: (0, 0)),
            pl.BlockSpec((1, 1), lambda i: (0, 0)),
            pl.BlockSpec((_BN, 1), lambda i: (i, 0)),
        ],
        out_specs=pl.BlockSpec((1, _NG), lambda i: (0, 0)),
        out_shape=jax.ShapeDtypeStruct((1, _NG), jnp.float32),
        scratch_shapes=[
            pltpu.VMEM((1, _NG), jnp.float32),
            pltpu.VMEM((1, _NG), jnp.float32),
        ],
    )(ag, h, dg, b, wfct, bfc, batch)


def kernel(x, edge_index, batch, W1, b1, W2, b2, Wfc, bfc):
    n, _ = x.shape
    hid = W1.shape[1]
    e = edge_index.shape[1]
    npad = _ceil_to((n + _NS - 1) // _NS, 8) * _NS
    if npad == n:
        npad += 8 * _NS
    ec = _ceil_to(e, _CH)
    nrows = _ceil_to(ec // _CH, _NW * 4)
    nck = nrows // _NW

    pad_e = nrows * _CH - e
    if pad_e:
        ar = jnp.arange(pad_e, dtype=jnp.int32)
        pad = jnp.stack([ar % n, n + ar % (npad - n)])
        eidx = jnp.concatenate([edge_index, pad.reshape(2, pad_e)], axis=1)
    else:
        eidx = edge_index
    eidx = eidx.reshape(2, nrows, _CH)

    zo = jnp.concatenate([jnp.zeros((npad, 16), jnp.float32),
                          jnp.ones((_CH, 16), jnp.float32)])
    degp = _make_deg(nck, npad)(eidx, zo)
    dg = (degp[0, :, 0] + degp[1, :, 0]).reshape(npad, 1)[:n]

    h1u = _matmul_tc(x, W1)
    h1 = _scale_tc(h1u, dg)
    ag1 = _make_agg(n, hid, nck, npad)(h1, eidx)
    h2 = _layer2_tc(ag1, h1, dg, b1.reshape(1, hid), W2)
    ag2 = _make_agg(n, hid, nck, npad)(h2, eidx)
    pooled = _final_tc(ag2, h2, dg, b2.reshape(1, hid),
                       Wfc.reshape(1, hid), bfc.reshape(1, 1),
                       batch.reshape(n, 1))
    return pooled.reshape(_NG, 1)
